# baseline (device time: 19775 ns/iter reference)
import jax
import jax.numpy as jnp
from jax import lax
from jax.experimental import pallas as pl
from jax.experimental.pallas import tpu as pltpu

_EPS = 1e-5
_BM = 512


def kernel(x, dy, gamma):
    del gamma
    m, d = x.shape
    half = m // 2
    nb = half // _BM

    def body(off_ref, x_ref, dy_ref, out_ref,
             acc_ref, last_ref, recv_early, recv_last, send_sems, recv_sems):
        i = pl.program_id(0)

        xv = x_ref[:, :]
        dyv = dy_ref[:, :]

        bm = xv.shape[0]
        dd = xv.shape[1]
        ones_d = jnp.ones((dd, 1), jnp.float32)
        row_x = lax.dot_general(
            xv, ones_d, (((1,), (0,)), ((), ())))
        row_xx = lax.dot_general(
            xv * xv, ones_d, (((1,), (0,)), ((), ())))
        mu = row_x / dd
        var = row_xx / dd - mu * mu
        a = lax.rsqrt(var + _EPS)
        b = mu * a

        p = xv * dyv
        s1 = lax.dot_general(
            a, p, (((0,), (0,)), ((), ())))
        w2 = jnp.concatenate([jnp.ones((bm, 1), jnp.float32), b], axis=1)
        s2 = lax.dot_general(
            w2, dyv, (((0,), (0,)), ((), ())))
        dgamma = s1 - s2[1:2, :]
        part = jnp.concatenate([dgamma, s2[0:1, :]], axis=0)

        my_x = lax.axis_index("x")
        my_y = lax.axis_index("y")
        peers = [
            (1 - my_x, my_y),
            (my_x, 1 - my_y),
            (1 - my_x, 1 - my_y),
        ]

        def mk(phase, k, src, dst):
            return pltpu.make_async_remote_copy(
                src_ref=src,
                dst_ref=dst,
                send_sem=send_sems.at[phase, k],
                recv_sem=recv_sems.at[phase, k],
                device_id=peers[k],
                device_id_type=pl.DeviceIdType.MESH,
            )

        @pl.when(i == 0)
        def _():
            barrier = pltpu.get_barrier_semaphore()
            for p in peers:
                pl.semaphore_signal(
                    barrier, inc=1,
                    device_id=p, device_id_type=pl.DeviceIdType.MESH,
                )
            pl.semaphore_wait(barrier, 3)
            acc_ref[:, :] = part

        @pl.when(jnp.logical_and(i > 0, i <= nb - 2))
        def _():
            acc_ref[:, :] = acc_ref[:, :] + part

        @pl.when(i == nb - 2)
        def _():
            for k in range(3):
                mk(0, k, acc_ref, recv_early.at[k]).start()

        @pl.when(i == nb - 1)
        def _():
            last_ref[:, :] = part
            for k in range(3):
                mk(1, k, last_ref, recv_last.at[k]).start()
            total = acc_ref[:, :] + part
            for k in range(3):
                mk(0, k, acc_ref, recv_early.at[k]).wait_recv()
                mk(1, k, last_ref, recv_last.at[k]).wait_recv()
                total = total + recv_early[k, :, :] + recv_last[k, :, :]
            out_ref[:, :] = total
            for k in range(3):
                mk(0, k, acc_ref, recv_early.at[k]).wait_send()
                mk(1, k, last_ref, recv_last.at[k]).wait_send()

    grid_spec = pltpu.PrefetchScalarGridSpec(
        num_scalar_prefetch=1,
        grid=(nb,),
        in_specs=[
            pl.BlockSpec((_BM, d), lambda i, off: (off[0] + i, 0)),
            pl.BlockSpec((_BM, d), lambda i, off: (off[0] + i, 0)),
        ],
        out_specs=pl.BlockSpec((2, d), lambda i, off: (0, 0)),
        scratch_shapes=[
            pltpu.VMEM((2, d), jnp.float32),
            pltpu.VMEM((2, d), jnp.float32),
            pltpu.VMEM((3, 2, d), jnp.float32),
            pltpu.VMEM((3, 2, d), jnp.float32),
            pltpu.SemaphoreType.DMA((2, 3)),
            pltpu.SemaphoreType.DMA((2, 3)),
        ],
    )

    block_off = (lax.axis_index("x") * nb).astype(jnp.int32).reshape((1,))

    return pl.pallas_call(
        body,
        grid_spec=grid_spec,
        out_shape=jax.ShapeDtypeStruct((2, d), jnp.float32),
        compiler_params=pltpu.CompilerParams(
            collective_id=0,
            dimension_semantics=("arbitrary",),
        ),
    )(block_off, x, dy)


# device time: 19090 ns/iter; 1.0359x vs baseline; 1.0359x over previous
import jax
import jax.numpy as jnp
from jax import lax
from jax.experimental import pallas as pl
from jax.experimental.pallas import tpu as pltpu

_EPS = 1e-5
_BM = 256


def kernel(x, dy, gamma):
    del gamma
    m, d = x.shape
    half = m // 2
    nb = half // _BM

    def body(off_ref, x_ref, dy_ref, out_ref,
             acc_ref, last_ref, recv_early, recv_last, send_sems, recv_sems):
        i = pl.program_id(0)

        xv = x_ref[:, :]
        dyv = dy_ref[:, :]
        mu = jnp.mean(xv, axis=1, keepdims=True)
        xc = xv - mu
        var = jnp.mean(xc * xc, axis=1, keepdims=True)
        xhat = xc * lax.rsqrt(var + _EPS)
        dgamma = jnp.sum(dyv * xhat, axis=0, keepdims=True)
        dbeta = jnp.sum(dyv, axis=0, keepdims=True)
        part = jnp.concatenate([dgamma, dbeta], axis=0)

        my_x = lax.axis_index("x")
        my_y = lax.axis_index("y")
        peers = [
            (1 - my_x, my_y),
            (my_x, 1 - my_y),
            (1 - my_x, 1 - my_y),
        ]

        def mk(phase, k, src, dst):
            return pltpu.make_async_remote_copy(
                src_ref=src,
                dst_ref=dst,
                send_sem=send_sems.at[phase, k],
                recv_sem=recv_sems.at[phase, k],
                device_id=peers[k],
                device_id_type=pl.DeviceIdType.MESH,
            )

        @pl.when(i == 0)
        def _():
            barrier = pltpu.get_barrier_semaphore()
            for p in peers:
                pl.semaphore_signal(
                    barrier, inc=1,
                    device_id=p, device_id_type=pl.DeviceIdType.MESH,
                )
            pl.semaphore_wait(barrier, 3)
            acc_ref[:, :] = part

        @pl.when(jnp.logical_and(i > 0, i <= nb - 2))
        def _():
            acc_ref[:, :] = acc_ref[:, :] + part

        @pl.when(i == nb - 2)
        def _():
            for k in range(3):
                mk(0, k, acc_ref, recv_early.at[k]).start()

        @pl.when(i == nb - 1)
        def _():
            last_ref[:, :] = part
            for k in range(3):
                mk(1, k, last_ref, recv_last.at[k]).start()
            total = acc_ref[:, :] + part
            for k in range(3):
                mk(0, k, acc_ref, recv_early.at[k]).wait_recv()
                mk(1, k, last_ref, recv_last.at[k]).wait_recv()
                total = total + recv_early[k, :, :] + recv_last[k, :, :]
            out_ref[:, :] = total
            for k in range(3):
                mk(0, k, acc_ref, recv_early.at[k]).wait_send()
                mk(1, k, last_ref, recv_last.at[k]).wait_send()

    grid_spec = pltpu.PrefetchScalarGridSpec(
        num_scalar_prefetch=1,
        grid=(nb,),
        in_specs=[
            pl.BlockSpec((_BM, d), lambda i, off: (off[0] + i, 0)),
            pl.BlockSpec((_BM, d), lambda i, off: (off[0] + i, 0)),
        ],
        out_specs=pl.BlockSpec((2, d), lambda i, off: (0, 0)),
        scratch_shapes=[
            pltpu.VMEM((2, d), jnp.float32),
            pltpu.VMEM((2, d), jnp.float32),
            pltpu.VMEM((3, 2, d), jnp.float32),
            pltpu.VMEM((3, 2, d), jnp.float32),
            pltpu.SemaphoreType.DMA((2, 3)),
            pltpu.SemaphoreType.DMA((2, 3)),
        ],
    )

    block_off = (lax.axis_index("x") * nb).astype(jnp.int32).reshape((1,))

    return pl.pallas_call(
        body,
        grid_spec=grid_spec,
        out_shape=jax.ShapeDtypeStruct((2, d), jnp.float32),
        compiler_params=pltpu.CompilerParams(
            collective_id=0,
            dimension_semantics=("arbitrary",),
        ),
    )(block_off, x, dy)


# device time: 17772 ns/iter; 1.1127x vs baseline; 1.0742x over previous
import jax
import jax.numpy as jnp
from jax import lax
from jax.experimental import pallas as pl
from jax.experimental.pallas import tpu as pltpu

_EPS = 1e-5
_BM = 512


def kernel(x, dy, gamma):
    del gamma
    m, d = x.shape
    half = m // 2
    nb = half // _BM

    def body(off_ref, x_ref, dy_ref, out_ref,
             acc_ref, last_ref, recv_early, recv_last, send_sems, recv_sems):
        i = pl.program_id(0)

        xv = x_ref[:, :]
        dyv = dy_ref[:, :]
        mu = jnp.mean(xv, axis=1, keepdims=True)
        xc = xv - mu
        var = jnp.mean(xc * xc, axis=1, keepdims=True)
        xhat = xc * lax.rsqrt(var + _EPS)
        dgamma = jnp.sum(dyv * xhat, axis=0, keepdims=True)
        dbeta = jnp.sum(dyv, axis=0, keepdims=True)
        part = jnp.concatenate([dgamma, dbeta], axis=0)

        my_x = lax.axis_index("x")
        my_y = lax.axis_index("y")
        peers = [
            (1 - my_x, my_y),
            (my_x, 1 - my_y),
            (1 - my_x, 1 - my_y),
        ]

        def mk(phase, k, src, dst):
            return pltpu.make_async_remote_copy(
                src_ref=src,
                dst_ref=dst,
                send_sem=send_sems.at[phase, k],
                recv_sem=recv_sems.at[phase, k],
                device_id=peers[k],
                device_id_type=pl.DeviceIdType.MESH,
            )

        @pl.when(i == 0)
        def _():
            barrier = pltpu.get_barrier_semaphore()
            for p in peers:
                pl.semaphore_signal(
                    barrier, inc=1,
                    device_id=p, device_id_type=pl.DeviceIdType.MESH,
                )
            acc_ref[:, :] = part

        @pl.when(jnp.logical_and(i > 0, i <= nb - 2))
        def _():
            acc_ref[:, :] = acc_ref[:, :] + part

        @pl.when(i == nb - 2)
        def _():
            pl.semaphore_wait(pltpu.get_barrier_semaphore(), 3)
            for k in range(3):
                mk(0, k, acc_ref, recv_early.at[k]).start()

        @pl.when(i == nb - 1)
        def _():
            last_ref[:, :] = part
            for k in range(3):
                mk(1, k, last_ref, recv_last.at[k]).start()
            total = acc_ref[:, :] + part
            for k in range(3):
                mk(0, k, acc_ref, recv_early.at[k]).wait_recv()
                total = total + recv_early[k, :, :]
            for k in range(3):
                mk(1, k, last_ref, recv_last.at[k]).wait_recv()
                total = total + recv_last[k, :, :]
            out_ref[:, :] = total
            for k in range(3):
                mk(0, k, acc_ref, recv_early.at[k]).wait_send()
                mk(1, k, last_ref, recv_last.at[k]).wait_send()

    grid_spec = pltpu.PrefetchScalarGridSpec(
        num_scalar_prefetch=1,
        grid=(nb,),
        in_specs=[
            pl.BlockSpec((_BM, d), lambda i, off: (off[0] + i, 0)),
            pl.BlockSpec((_BM, d), lambda i, off: (off[0] + i, 0)),
        ],
        out_specs=pl.BlockSpec((2, d), lambda i, off: (0, 0)),
        scratch_shapes=[
            pltpu.VMEM((2, d), jnp.float32),
            pltpu.VMEM((2, d), jnp.float32),
            pltpu.VMEM((3, 2, d), jnp.float32),
            pltpu.VMEM((3, 2, d), jnp.float32),
            pltpu.SemaphoreType.DMA((2, 3)),
            pltpu.SemaphoreType.DMA((2, 3)),
        ],
    )

    block_off = (lax.axis_index("x") * nb).astype(jnp.int32).reshape((1,))

    return pl.pallas_call(
        body,
        grid_spec=grid_spec,
        out_shape=jax.ShapeDtypeStruct((2, d), jnp.float32),
        compiler_params=pltpu.CompilerParams(
            collective_id=0,
            dimension_semantics=("arbitrary",),
        ),
    )(block_off, x, dy)
